# Initial kernel scaffold; baseline (speedup 1.0000x reference)
import jax
import jax.numpy as jnp
from jax.experimental import pallas as pl


def kernel(x_user, x_item, ei_clicks, ei_clicked_by, Wp_user, bp_user, Wp_item, bp_item, Ws0_clicks, Wn0_clicks, b0_clicks, Ws0_cb, Wn0_cb, b0_cb, Ws1_clicks, Wn1_clicks, b1_clicks, Ws1_cb, Wn1_cb, b1_cb):
    raise NotImplementedError("write your pallas kernel here")


# speedup vs baseline: 3.0063x; 3.0063x over previous
import functools

import jax
import jax.numpy as jnp
from jax import lax
from jax.experimental import pallas as pl
from jax.experimental.pallas import tpu as pltpu
from jax.experimental.pallas import tpu_sc as plsc

N = 10000
E = 320000
D = 128
NS = 16
CHUNK = 80
EPT = E // NS
NCH = EPT // CHUNK
RPT = 640
RCH = 160
DEGW = 128


def _make_sc_agg(compute_deg):
    outs = [jax.ShapeDtypeStruct((N, D), jnp.float32),
            jax.ShapeDtypeStruct((N, D), jnp.float32)]
    scratch = [
        pltpu.VMEM((CHUNK,), jnp.int32),
        pltpu.VMEM((CHUNK,), jnp.int32),
        pltpu.VMEM((CHUNK, D), jnp.float32),
        pltpu.VMEM((RCH, D), jnp.float32),
        pltpu.VMEM_SHARED((N, D), jnp.float32),
        pltpu.SemaphoreType.DMA,
    ]
    if compute_deg:
        outs += [jax.ShapeDtypeStruct((N, DEGW), jnp.float32),
                 jax.ShapeDtypeStruct((N, DEGW), jnp.float32)]
        scratch += [
            pltpu.VMEM((CHUNK, DEGW), jnp.float32),
            pltpu.VMEM((RCH, DEGW), jnp.float32),
            pltpu.VMEM_SHARED((N, DEGW), jnp.float32),
        ]

    mesh = plsc.VectorSubcoreMesh(core_axis_name="c", subcore_axis_name="s")

    @functools.partial(pl.kernel, out_type=tuple(outs), mesh=mesh,
                       scratch_types=scratch)
    def sc_agg(*refs):
        if compute_deg:
            (hu, hi, src_ck, dst_ck, src_cb, dst_cb, zD, zW, onesW,
             summ_u, summ_i, deg_u, deg_i,
             idx_s, idx_d, rows_v, stage_v, acc_sh, sem,
             ones_v, stageW_v, degacc_sh) = refs
        else:
            (hu, hi, src_ck, dst_ck, src_cb, dst_cb, zD,
             summ_u, summ_i,
             idx_s, idx_d, rows_v, stage_v, acc_sh, sem) = refs

        c = lax.axis_index("c")
        s = lax.axis_index("s")
        r0 = jnp.minimum(s * RPT, N - RPT)
        e0 = s * EPT

        def do_rel(feat_hbm, src_hbm, dst_hbm, summ_hbm, deg_hbm):
            pltpu.sync_copy(zD, stage_v)
            if compute_deg:
                pltpu.sync_copy(zW, stageW_v)
                pltpu.sync_copy(onesW, ones_v)

            def zero_chunk(k, carry):
                pltpu.sync_copy(stage_v, acc_sh.at[pl.ds(r0 + k * RCH, RCH), :])
                if compute_deg:
                    pltpu.sync_copy(stageW_v,
                                    degacc_sh.at[pl.ds(r0 + k * RCH, RCH), :])
                return carry

            lax.fori_loop(0, RPT // RCH, zero_chunk, 0)
            plsc.subcore_barrier()

            def chunk(i, carry):
                base = e0 + i * CHUNK
                pltpu.sync_copy(src_hbm.at[pl.ds(base, CHUNK)], idx_s)
                pltpu.sync_copy(dst_hbm.at[pl.ds(base, CHUNK)], idx_d)
                pltpu.async_copy(feat_hbm.at[idx_s], rows_v, sem).wait()
                pltpu.sync_copy(rows_v, acc_sh.at[idx_d], add=True)
                if compute_deg:
                    pltpu.sync_copy(ones_v, degacc_sh.at[idx_d], add=True)
                return carry

            lax.fori_loop(0, NCH, chunk, 0)
            plsc.subcore_barrier()

            def out_chunk(k, carry):
                rb = r0 + k * RCH
                pltpu.sync_copy(acc_sh.at[pl.ds(rb, RCH), :], stage_v)
                pltpu.sync_copy(stage_v, summ_hbm.at[pl.ds(rb, RCH), :])
                if compute_deg:
                    pltpu.sync_copy(degacc_sh.at[pl.ds(rb, RCH), :], stageW_v)
                    pltpu.sync_copy(stageW_v, deg_hbm.at[pl.ds(rb, RCH), :])
                return carry

            lax.fori_loop(0, RPT // RCH, out_chunk, 0)

        @pl.when(c == 0)
        def _():
            do_rel(hi, src_cb, dst_cb, summ_u,
                   deg_u if compute_deg else None)

        @pl.when(c == 1)
        def _():
            do_rel(hu, src_ck, dst_ck, summ_i,
                   deg_i if compute_deg else None)

    return sc_agg


_sc_agg0 = _make_sc_agg(False)
_sc_agg1 = _make_sc_agg(False)


def _make_sc_deg():
    outs = (jax.ShapeDtypeStruct((N, DEGW), jnp.float32),
            jax.ShapeDtypeStruct((N, DEGW), jnp.float32))
    scratch = [
        pltpu.VMEM((CHUNK,), jnp.int32),
        pltpu.VMEM((CHUNK, DEGW), jnp.float32),
        pltpu.VMEM((RCH, DEGW), jnp.float32),
        pltpu.VMEM_SHARED((N, DEGW), jnp.float32),
    ]
    mesh = plsc.VectorSubcoreMesh(core_axis_name="c", subcore_axis_name="s")

    @functools.partial(pl.kernel, out_type=outs, mesh=mesh,
                       scratch_types=scratch)
    def sc_deg(dst_ck, dst_cb, zW, onesW, deg_u, deg_i,
               idx_d, ones_v, stageW_v, degacc_sh):
        c = lax.axis_index("c")
        s = lax.axis_index("s")
        r0 = jnp.minimum(s * RPT, N - RPT)
        e0 = s * EPT

        def do_rel(dst_hbm, deg_hbm):
            pltpu.sync_copy(zW, stageW_v)
            pltpu.sync_copy(onesW, ones_v)

            def zero_chunk(k, carry):
                pltpu.sync_copy(stageW_v,
                                degacc_sh.at[pl.ds(r0 + k * RCH, RCH), :])
                return carry

            lax.fori_loop(0, RPT // RCH, zero_chunk, 0)
            plsc.subcore_barrier()

            def chunk(i, carry):
                base = e0 + i * CHUNK
                pltpu.sync_copy(dst_hbm.at[pl.ds(base, CHUNK)], idx_d)
                pltpu.sync_copy(ones_v, degacc_sh.at[idx_d], add=True)
                return carry

            lax.fori_loop(0, NCH, chunk, 0)
            plsc.subcore_barrier()

            def out_chunk(k, carry):
                rb = r0 + k * RCH
                pltpu.sync_copy(degacc_sh.at[pl.ds(rb, RCH), :], stageW_v)
                pltpu.sync_copy(stageW_v, deg_hbm.at[pl.ds(rb, RCH), :])
                return carry

            lax.fori_loop(0, RPT // RCH, out_chunk, 0)

        @pl.when(c == 0)
        def _():
            do_rel(dst_cb, deg_u)

        @pl.when(c == 1)
        def _():
            do_rel(dst_ck, deg_i)

    return sc_deg


_sc_deg = _make_sc_deg()


_BLK = 1000


def _proj_body(x_ref, w_ref, b_ref, o_ref):
    y = jnp.dot(x_ref[...], w_ref[...], preferred_element_type=jnp.float32,
                precision=lax.Precision.HIGHEST)
    o_ref[...] = jnp.maximum(y + b_ref[0:1, :], 0.0)


def _proj(x, w, b):
    return pl.pallas_call(
        _proj_body,
        grid=(N // _BLK,),
        in_specs=[pl.BlockSpec((_BLK, D), lambda i: (i, 0)),
                  pl.BlockSpec((D, D), lambda i: (0, 0)),
                  pl.BlockSpec((8, D), lambda i: (0, 0))],
        out_specs=pl.BlockSpec((_BLK, D), lambda i: (i, 0)),
        out_shape=jax.ShapeDtypeStruct((N, D), jnp.float32),
    )(x, w, jnp.broadcast_to(b[None, :], (8, D)))


def _comb_body(norm, h_ref, s_ref, d_ref, ws_ref, wn_ref, b_ref, o_ref):
    deg = jnp.maximum(d_ref[...][:, :1], 1.0)
    hn = s_ref[...] / deg
    y = (jnp.dot(h_ref[...], ws_ref[...], preferred_element_type=jnp.float32,
                 precision=lax.Precision.HIGHEST)
         + jnp.dot(hn, wn_ref[...], preferred_element_type=jnp.float32,
                   precision=lax.Precision.HIGHEST)
         + b_ref[0:1, :])
    y = jnp.maximum(y, 0.0)
    if norm:
        nrm = jnp.sqrt(jnp.sum(y * y, axis=1, keepdims=True))
        y = y / jnp.maximum(nrm, 1e-12)
    o_ref[...] = y


def _comb(h, summ, deg, ws, wn, b, norm):
    return pl.pallas_call(
        functools.partial(_comb_body, norm),
        grid=(N // _BLK,),
        in_specs=[pl.BlockSpec((_BLK, D), lambda i: (i, 0)),
                  pl.BlockSpec((_BLK, D), lambda i: (i, 0)),
                  pl.BlockSpec((_BLK, DEGW), lambda i: (i, 0)),
                  pl.BlockSpec((D, D), lambda i: (0, 0)),
                  pl.BlockSpec((D, D), lambda i: (0, 0)),
                  pl.BlockSpec((8, D), lambda i: (0, 0))],
        out_specs=pl.BlockSpec((_BLK, D), lambda i: (i, 0)),
        out_shape=jax.ShapeDtypeStruct((N, D), jnp.float32),
    )(h, summ, deg, ws, wn, jnp.broadcast_to(b[None, :], (8, D)))


def kernel(x_user, x_item, ei_clicks, ei_clicked_by, Wp_user, bp_user,
           Wp_item, bp_item, Ws0_clicks, Wn0_clicks, b0_clicks, Ws0_cb,
           Wn0_cb, b0_cb, Ws1_clicks, Wn1_clicks, b1_clicks, Ws1_cb,
           Wn1_cb, b1_cb):
    ei_ck = ei_clicks.astype(jnp.int32)
    ei_cb = ei_clicked_by.astype(jnp.int32)
    src_ck, dst_ck = ei_ck[0], ei_ck[1]
    src_cb, dst_cb = ei_cb[0], ei_cb[1]
    zD = jnp.zeros((RCH, D), jnp.float32)
    zW = jnp.zeros((RCH, DEGW), jnp.float32)
    onesW = jnp.ones((CHUNK, DEGW), jnp.float32)

    h_u = _proj(x_user, Wp_user, bp_user)
    h_i = _proj(x_item, Wp_item, bp_item)

    deg_u, deg_i = _sc_deg(dst_ck, dst_cb, zW, onesW)
    summ_u, summ_i = _sc_agg0(h_u, h_i, src_ck, dst_ck,
                              src_cb, dst_cb, zD)
    h_u = _comb(h_u, summ_u, deg_u, Ws0_cb, Wn0_cb, b0_cb, norm=False)
    h_i = _comb(h_i, summ_i, deg_i, Ws0_clicks, Wn0_clicks, b0_clicks,
                norm=False)

    summ_u, summ_i = _sc_agg1(h_u, h_i, src_ck, dst_ck, src_cb, dst_cb, zD)
    h_u = _comb(h_u, summ_u, deg_u, Ws1_cb, Wn1_cb, b1_cb, norm=True)
    h_i = _comb(h_i, summ_i, deg_i, Ws1_clicks, Wn1_clicks, b1_clicks,
                norm=True)
    return (h_u, h_i)


# speedup vs baseline: 5.3767x; 1.7885x over previous
import functools

import jax
import jax.numpy as jnp
from jax import lax
from jax.experimental import pallas as pl
from jax.experimental.pallas import tpu as pltpu
from jax.experimental.pallas import tpu_sc as plsc

N = 10000
E = 320000
D = 128
NS = 16
CHUNK = 80
EPP = E // NS
NCH = EPP // CHUNK
NPAIR = NCH // 2
RPT = 640
RCH = 160
DEGW = 128

_MESH = dict(core_axis_name="c", subcore_axis_name="s")


def _make_sc_agg():
    outs = (jax.ShapeDtypeStruct((N, D), jnp.float32),
            jax.ShapeDtypeStruct((N, D), jnp.float32))
    scratch = [
        pltpu.VMEM((CHUNK,), jnp.int32),
        pltpu.VMEM((CHUNK,), jnp.int32),
        pltpu.VMEM((CHUNK,), jnp.int32),
        pltpu.VMEM((CHUNK,), jnp.int32),
        pltpu.VMEM((CHUNK, D), jnp.float32),
        pltpu.VMEM((CHUNK, D), jnp.float32),
        pltpu.VMEM((RCH, D), jnp.float32),
        pltpu.VMEM_SHARED((N, D), jnp.float32),
        pltpu.SemaphoreType.DMA,
        pltpu.SemaphoreType.DMA,
        pltpu.SemaphoreType.DMA,
        pltpu.SemaphoreType.DMA,
    ]

    @functools.partial(pl.kernel, out_type=outs,
                       mesh=plsc.VectorSubcoreMesh(**_MESH),
                       scratch_types=scratch)
    def sc_agg(hu, hi, src_ck, dst_ck, src_cb, dst_cb, zD,
               summ_u, summ_i,
               is0, is1, id0, id1, rows0, rows1, stage_v, acc_sh,
               semi0, semi1, semg0, semg1):
        c = lax.axis_index("c")
        s = lax.axis_index("s")
        r0 = jnp.minimum(s * RPT, N - RPT)
        e0 = s * EPP

        def do_rel(feat_hbm, src_hbm, dst_hbm, summ_hbm):
            pltpu.sync_copy(zD, stage_v)

            def zero_chunk(k, carry):
                pltpu.sync_copy(stage_v, acc_sh.at[pl.ds(r0 + k * RCH, RCH), :])
                return carry

            lax.fori_loop(0, RPT // RCH, zero_chunk, 0)
            plsc.subcore_barrier()

            def idx_start(ci, bs, bd, sem):
                base = e0 + ci * CHUNK
                pltpu.async_copy(src_hbm.at[pl.ds(base, CHUNK)], bs, sem)
                pltpu.async_copy(dst_hbm.at[pl.ds(base, CHUNK)], bd, sem)

            def idx_wait(ci, bs, bd, sem):
                base = e0 + ci * CHUNK
                pltpu.make_async_copy(src_hbm.at[pl.ds(base, CHUNK)], bs,
                                      sem).wait()
                pltpu.make_async_copy(dst_hbm.at[pl.ds(base, CHUNK)], bd,
                                      sem).wait()

            def gather_start(bs, rows, sem):
                pltpu.async_copy(feat_hbm.at[bs], rows, sem)

            def gather_wait(bs, rows, sem):
                pltpu.make_async_copy(feat_hbm.at[bs], rows, sem).wait()

            def scatter(bd, rows):
                pltpu.sync_copy(rows, acc_sh.at[bd], add=True)

            idx_start(0, is0, id0, semi0)
            idx_start(1, is1, id1, semi1)
            idx_wait(0, is0, id0, semi0)
            gather_start(is0, rows0, semg0)
            idx_wait(1, is1, id1, semi1)

            def pair(j, carry):
                c0 = 2 * j
                gather_start(is1, rows1, semg1)
                gather_wait(is0, rows0, semg0)
                scatter(id0, rows0)
                idx_start(c0 + 2, is0, id0, semi0)
                gather_wait(is1, rows1, semg1)
                scatter(id1, rows1)
                idx_start(c0 + 3, is1, id1, semi1)
                idx_wait(c0 + 2, is0, id0, semi0)
                gather_start(is0, rows0, semg0)
                idx_wait(c0 + 3, is1, id1, semi1)
                return carry

            lax.fori_loop(0, NPAIR - 1, pair, 0)
            gather_start(is1, rows1, semg1)
            gather_wait(is0, rows0, semg0)
            scatter(id0, rows0)
            gather_wait(is1, rows1, semg1)
            scatter(id1, rows1)

            plsc.subcore_barrier()

            def out_chunk(k, carry):
                rb = r0 + k * RCH
                pltpu.sync_copy(acc_sh.at[pl.ds(rb, RCH), :], stage_v)
                pltpu.sync_copy(stage_v, summ_hbm.at[pl.ds(rb, RCH), :])
                return carry

            lax.fori_loop(0, RPT // RCH, out_chunk, 0)

        @pl.when(c == 0)
        def _():
            do_rel(hi, src_cb, dst_cb, summ_u)

        @pl.when(c == 1)
        def _():
            do_rel(hu, src_ck, dst_ck, summ_i)

    return sc_agg


_sc_agg = _make_sc_agg()


def _make_sc_deg():
    outs = (jax.ShapeDtypeStruct((N, DEGW), jnp.float32),
            jax.ShapeDtypeStruct((N, DEGW), jnp.float32))
    scratch = [
        pltpu.VMEM((CHUNK,), jnp.int32),
        pltpu.VMEM((CHUNK,), jnp.int32),
        pltpu.VMEM((CHUNK, DEGW), jnp.float32),
        pltpu.VMEM((RCH, DEGW), jnp.float32),
        pltpu.VMEM_SHARED((N, DEGW), jnp.float32),
        pltpu.SemaphoreType.DMA,
        pltpu.SemaphoreType.DMA,
    ]

    @functools.partial(pl.kernel, out_type=outs,
                       mesh=plsc.VectorSubcoreMesh(**_MESH),
                       scratch_types=scratch)
    def sc_deg(dst_ck, dst_cb, zW, onesW, deg_u, deg_i,
               id0, id1, ones_v, stageW_v, degacc_sh, semi0, semi1):
        c = lax.axis_index("c")
        s = lax.axis_index("s")
        r0 = jnp.minimum(s * RPT, N - RPT)
        e0 = s * EPP

        def do_rel(dst_hbm, deg_hbm):
            pltpu.sync_copy(zW, stageW_v)
            pltpu.sync_copy(onesW, ones_v)

            def zero_chunk(k, carry):
                pltpu.sync_copy(stageW_v,
                                degacc_sh.at[pl.ds(r0 + k * RCH, RCH), :])
                return carry

            lax.fori_loop(0, RPT // RCH, zero_chunk, 0)
            plsc.subcore_barrier()

            def idx_start(ci, bd, sem):
                pltpu.async_copy(dst_hbm.at[pl.ds(e0 + ci * CHUNK, CHUNK)],
                                 bd, sem)

            def idx_wait(ci, bd, sem):
                pltpu.make_async_copy(
                    dst_hbm.at[pl.ds(e0 + ci * CHUNK, CHUNK)], bd, sem).wait()

            def scatter(bd):
                pltpu.sync_copy(ones_v, degacc_sh.at[bd], add=True)

            idx_start(0, id0, semi0)
            idx_start(1, id1, semi1)
            idx_wait(0, id0, semi0)
            idx_wait(1, id1, semi1)

            def pair(j, carry):
                c0 = 2 * j
                scatter(id0)
                idx_start(c0 + 2, id0, semi0)
                scatter(id1)
                idx_start(c0 + 3, id1, semi1)
                idx_wait(c0 + 2, id0, semi0)
                idx_wait(c0 + 3, id1, semi1)
                return carry

            lax.fori_loop(0, NPAIR - 1, pair, 0)
            scatter(id0)
            scatter(id1)
            plsc.subcore_barrier()

            def out_chunk(k, carry):
                rb = r0 + k * RCH
                pltpu.sync_copy(degacc_sh.at[pl.ds(rb, RCH), :], stageW_v)
                pltpu.sync_copy(stageW_v, deg_hbm.at[pl.ds(rb, RCH), :])
                return carry

            lax.fori_loop(0, RPT // RCH, out_chunk, 0)

        @pl.when(c == 0)
        def _():
            do_rel(dst_cb, deg_u)

        @pl.when(c == 1)
        def _():
            do_rel(dst_ck, deg_i)

    return sc_deg


_sc_deg = _make_sc_deg()


_BLK = 1000


def _proj_body(x_ref, w_ref, b_ref, o_ref):
    y = jnp.dot(x_ref[...], w_ref[...], preferred_element_type=jnp.float32,
                precision=lax.Precision.HIGHEST)
    o_ref[...] = jnp.maximum(y + b_ref[0:1, :], 0.0)


def _proj(x, w, b):
    return pl.pallas_call(
        _proj_body,
        grid=(N // _BLK,),
        in_specs=[pl.BlockSpec((_BLK, D), lambda i: (i, 0)),
                  pl.BlockSpec((D, D), lambda i: (0, 0)),
                  pl.BlockSpec((8, D), lambda i: (0, 0))],
        out_specs=pl.BlockSpec((_BLK, D), lambda i: (i, 0)),
        out_shape=jax.ShapeDtypeStruct((N, D), jnp.float32),
    )(x, w, jnp.broadcast_to(b[None, :], (8, D)))


def _comb_body(norm, h_ref, s_ref, d_ref, ws_ref, wn_ref, b_ref, o_ref):
    deg = jnp.maximum(d_ref[...][:, :1], 1.0)
    hn = s_ref[...] / deg
    y = (jnp.dot(h_ref[...], ws_ref[...], preferred_element_type=jnp.float32,
                 precision=lax.Precision.HIGHEST)
         + jnp.dot(hn, wn_ref[...], preferred_element_type=jnp.float32,
                   precision=lax.Precision.HIGHEST)
         + b_ref[0:1, :])
    y = jnp.maximum(y, 0.0)
    if norm:
        nrm = jnp.sqrt(jnp.sum(y * y, axis=1, keepdims=True))
        y = y / jnp.maximum(nrm, 1e-12)
    o_ref[...] = y


def _comb(h, summ, deg, ws, wn, b, norm):
    return pl.pallas_call(
        functools.partial(_comb_body, norm),
        grid=(N // _BLK,),
        in_specs=[pl.BlockSpec((_BLK, D), lambda i: (i, 0)),
                  pl.BlockSpec((_BLK, D), lambda i: (i, 0)),
                  pl.BlockSpec((_BLK, DEGW), lambda i: (i, 0)),
                  pl.BlockSpec((D, D), lambda i: (0, 0)),
                  pl.BlockSpec((D, D), lambda i: (0, 0)),
                  pl.BlockSpec((8, D), lambda i: (0, 0))],
        out_specs=pl.BlockSpec((_BLK, D), lambda i: (i, 0)),
        out_shape=jax.ShapeDtypeStruct((N, D), jnp.float32),
    )(h, summ, deg, ws, wn, jnp.broadcast_to(b[None, :], (8, D)))


def kernel(x_user, x_item, ei_clicks, ei_clicked_by, Wp_user, bp_user,
           Wp_item, bp_item, Ws0_clicks, Wn0_clicks, b0_clicks, Ws0_cb,
           Wn0_cb, b0_cb, Ws1_clicks, Wn1_clicks, b1_clicks, Ws1_cb,
           Wn1_cb, b1_cb):
    ei_ck = ei_clicks.astype(jnp.int32)
    ei_cb = ei_clicked_by.astype(jnp.int32)
    src_ck, dst_ck = ei_ck[0], ei_ck[1]
    src_cb, dst_cb = ei_cb[0], ei_cb[1]
    zD = jnp.zeros((RCH, D), jnp.float32)
    zW = jnp.zeros((RCH, DEGW), jnp.float32)
    onesW = jnp.ones((CHUNK, DEGW), jnp.float32)

    h_u = _proj(x_user, Wp_user, bp_user)
    h_i = _proj(x_item, Wp_item, bp_item)

    deg_u, deg_i = _sc_deg(dst_ck, dst_cb, zW, onesW)
    summ_u, summ_i = _sc_agg(h_u, h_i, src_ck, dst_ck, src_cb, dst_cb, zD)
    h_u = _comb(h_u, summ_u, deg_u, Ws0_cb, Wn0_cb, b0_cb, norm=False)
    h_i = _comb(h_i, summ_i, deg_i, Ws0_clicks, Wn0_clicks, b0_clicks,
                norm=False)

    summ_u, summ_i = _sc_agg(h_u, h_i, src_ck, dst_ck, src_cb, dst_cb, zD)
    h_u = _comb(h_u, summ_u, deg_u, Ws1_cb, Wn1_cb, b1_cb, norm=True)
    h_i = _comb(h_i, summ_i, deg_i, Ws1_clicks, Wn1_clicks, b1_clicks,
                norm=True)
    return (h_u, h_i)


# speedup vs baseline: 5.6321x; 1.0475x over previous
import functools

import jax
import jax.numpy as jnp
from jax import lax
from jax.experimental import pallas as pl
from jax.experimental.pallas import tpu as pltpu
from jax.experimental.pallas import tpu_sc as plsc

N = 10000
E = 320000
D = 128
NS = 16
CHUNK = 96
EPP = E // NS
NCH = EPP // CHUNK
TCH = EPP - NCH * CHUNK
NPAIR = NCH // 2
RPT = 640
RCH = 160
DEGW = 128

_MESH = dict(core_axis_name="c", subcore_axis_name="s")


def _make_sc_agg():
    outs = (jax.ShapeDtypeStruct((N, D), jnp.float32),
            jax.ShapeDtypeStruct((N, D), jnp.float32))
    scratch = [
        pltpu.VMEM((CHUNK,), jnp.int32),
        pltpu.VMEM((CHUNK,), jnp.int32),
        pltpu.VMEM((CHUNK,), jnp.int32),
        pltpu.VMEM((CHUNK,), jnp.int32),
        pltpu.VMEM((CHUNK, D), jnp.float32),
        pltpu.VMEM((CHUNK, D), jnp.float32),
        pltpu.VMEM((RCH, D), jnp.float32),
        pltpu.VMEM((TCH,), jnp.int32),
        pltpu.VMEM((TCH,), jnp.int32),
        pltpu.VMEM_SHARED((N, D), jnp.float32),
        pltpu.SemaphoreType.DMA,
        pltpu.SemaphoreType.DMA,
        pltpu.SemaphoreType.DMA,
        pltpu.SemaphoreType.DMA,
    ]

    @functools.partial(pl.kernel, out_type=outs,
                       mesh=plsc.VectorSubcoreMesh(**_MESH),
                       scratch_types=scratch)
    def sc_agg(hu, hi, src_ck, dst_ck, src_cb, dst_cb, zD,
               summ_u, summ_i,
               is0, is1, id0, id1, rows0, rows1, stage_v, ist, idt, acc_sh,
               semi0, semi1, semg0, semg1):
        c = lax.axis_index("c")
        s = lax.axis_index("s")
        r0 = jnp.minimum(s * RPT, N - RPT)
        e0 = s * EPP

        def do_rel(feat_hbm, src_hbm, dst_hbm, summ_hbm):
            pltpu.sync_copy(zD, stage_v)

            def zero_chunk(k, carry):
                pltpu.sync_copy(stage_v, acc_sh.at[pl.ds(r0 + k * RCH, RCH), :])
                return carry

            lax.fori_loop(0, RPT // RCH, zero_chunk, 0)
            plsc.subcore_barrier()

            def idx_start(ci, bs, bd, sem):
                base = e0 + ci * CHUNK
                pltpu.async_copy(src_hbm.at[pl.ds(base, CHUNK)], bs, sem)
                pltpu.async_copy(dst_hbm.at[pl.ds(base, CHUNK)], bd, sem)

            def idx_wait(ci, bs, bd, sem):
                base = e0 + ci * CHUNK
                pltpu.make_async_copy(src_hbm.at[pl.ds(base, CHUNK)], bs,
                                      sem).wait()
                pltpu.make_async_copy(dst_hbm.at[pl.ds(base, CHUNK)], bd,
                                      sem).wait()

            def gather_start(bs, rows, sem):
                pltpu.async_copy(feat_hbm.at[bs], rows, sem)

            def gather_wait(bs, rows, sem):
                pltpu.make_async_copy(feat_hbm.at[bs], rows, sem).wait()

            def scatter(bd, rows):
                pltpu.sync_copy(rows, acc_sh.at[bd], add=True)

            idx_start(0, is0, id0, semi0)
            idx_start(1, is1, id1, semi1)
            idx_wait(0, is0, id0, semi0)
            gather_start(is0, rows0, semg0)
            idx_wait(1, is1, id1, semi1)

            def pair(j, carry):
                c0 = 2 * j
                gather_start(is1, rows1, semg1)
                gather_wait(is0, rows0, semg0)
                scatter(id0, rows0)
                idx_start(c0 + 2, is0, id0, semi0)
                gather_wait(is1, rows1, semg1)
                scatter(id1, rows1)
                idx_start(c0 + 3, is1, id1, semi1)
                idx_wait(c0 + 2, is0, id0, semi0)
                gather_start(is0, rows0, semg0)
                idx_wait(c0 + 3, is1, id1, semi1)
                return carry

            lax.fori_loop(0, NPAIR - 1, pair, 0)
            gather_start(is1, rows1, semg1)
            gather_wait(is0, rows0, semg0)
            scatter(id0, rows0)
            gather_wait(is1, rows1, semg1)
            scatter(id1, rows1)

            tbase = e0 + NCH * CHUNK
            pltpu.sync_copy(src_hbm.at[pl.ds(tbase, TCH)], ist)
            pltpu.sync_copy(dst_hbm.at[pl.ds(tbase, TCH)], idt)
            rows_t = rows0.at[pl.ds(0, TCH), :]
            pltpu.async_copy(feat_hbm.at[ist], rows_t, semg0).wait()
            pltpu.sync_copy(rows_t, acc_sh.at[idt], add=True)

            plsc.subcore_barrier()

            def out_chunk(k, carry):
                rb = r0 + k * RCH
                pltpu.sync_copy(acc_sh.at[pl.ds(rb, RCH), :], stage_v)
                pltpu.sync_copy(stage_v, summ_hbm.at[pl.ds(rb, RCH), :])
                return carry

            lax.fori_loop(0, RPT // RCH, out_chunk, 0)

        @pl.when(c == 0)
        def _():
            do_rel(hi, src_cb, dst_cb, summ_u)

        @pl.when(c == 1)
        def _():
            do_rel(hu, src_ck, dst_ck, summ_i)

    return sc_agg


_sc_agg = _make_sc_agg()


def _make_sc_deg():
    outs = (jax.ShapeDtypeStruct((N, DEGW), jnp.float32),
            jax.ShapeDtypeStruct((N, DEGW), jnp.float32))
    scratch = [
        pltpu.VMEM((CHUNK,), jnp.int32),
        pltpu.VMEM((CHUNK,), jnp.int32),
        pltpu.VMEM((CHUNK, DEGW), jnp.float32),
        pltpu.VMEM((RCH, DEGW), jnp.float32),
        pltpu.VMEM((TCH,), jnp.int32),
        pltpu.VMEM_SHARED((N, DEGW), jnp.float32),
        pltpu.SemaphoreType.DMA,
        pltpu.SemaphoreType.DMA,
    ]

    @functools.partial(pl.kernel, out_type=outs,
                       mesh=plsc.VectorSubcoreMesh(**_MESH),
                       scratch_types=scratch)
    def sc_deg(dst_ck, dst_cb, zW, onesW, deg_u, deg_i,
               id0, id1, ones_v, stageW_v, idt, degacc_sh, semi0, semi1):
        c = lax.axis_index("c")
        s = lax.axis_index("s")
        r0 = jnp.minimum(s * RPT, N - RPT)
        e0 = s * EPP

        def do_rel(dst_hbm, deg_hbm):
            pltpu.sync_copy(zW, stageW_v)
            pltpu.sync_copy(onesW, ones_v)

            def zero_chunk(k, carry):
                pltpu.sync_copy(stageW_v,
                                degacc_sh.at[pl.ds(r0 + k * RCH, RCH), :])
                return carry

            lax.fori_loop(0, RPT // RCH, zero_chunk, 0)
            plsc.subcore_barrier()

            def idx_start(ci, bd, sem):
                pltpu.async_copy(dst_hbm.at[pl.ds(e0 + ci * CHUNK, CHUNK)],
                                 bd, sem)

            def idx_wait(ci, bd, sem):
                pltpu.make_async_copy(
                    dst_hbm.at[pl.ds(e0 + ci * CHUNK, CHUNK)], bd, sem).wait()

            def scatter(bd):
                pltpu.sync_copy(ones_v, degacc_sh.at[bd], add=True)

            idx_start(0, id0, semi0)
            idx_start(1, id1, semi1)
            idx_wait(0, id0, semi0)
            idx_wait(1, id1, semi1)

            def pair(j, carry):
                c0 = 2 * j
                scatter(id0)
                idx_start(c0 + 2, id0, semi0)
                scatter(id1)
                idx_start(c0 + 3, id1, semi1)
                idx_wait(c0 + 2, id0, semi0)
                idx_wait(c0 + 3, id1, semi1)
                return carry

            lax.fori_loop(0, NPAIR - 1, pair, 0)
            scatter(id0)
            scatter(id1)
            tbase = e0 + NCH * CHUNK
            pltpu.sync_copy(dst_hbm.at[pl.ds(tbase, TCH)], idt)
            pltpu.sync_copy(ones_v.at[pl.ds(0, TCH), :],
                            degacc_sh.at[idt], add=True)
            plsc.subcore_barrier()

            def out_chunk(k, carry):
                rb = r0 + k * RCH
                pltpu.sync_copy(degacc_sh.at[pl.ds(rb, RCH), :], stageW_v)
                pltpu.sync_copy(stageW_v, deg_hbm.at[pl.ds(rb, RCH), :])
                return carry

            lax.fori_loop(0, RPT // RCH, out_chunk, 0)

        @pl.when(c == 0)
        def _():
            do_rel(dst_cb, deg_u)

        @pl.when(c == 1)
        def _():
            do_rel(dst_ck, deg_i)

    return sc_deg


_sc_deg = _make_sc_deg()


_BLK = 1000


def _proj_body(x_ref, w_ref, b_ref, o_ref):
    y = jnp.dot(x_ref[...], w_ref[...], preferred_element_type=jnp.float32,
                precision=lax.Precision.HIGHEST)
    o_ref[...] = jnp.maximum(y + b_ref[0:1, :], 0.0)


def _proj(x, w, b):
    return pl.pallas_call(
        _proj_body,
        grid=(N // _BLK,),
        in_specs=[pl.BlockSpec((_BLK, D), lambda i: (i, 0)),
                  pl.BlockSpec((D, D), lambda i: (0, 0)),
                  pl.BlockSpec((8, D), lambda i: (0, 0))],
        out_specs=pl.BlockSpec((_BLK, D), lambda i: (i, 0)),
        out_shape=jax.ShapeDtypeStruct((N, D), jnp.float32),
    )(x, w, jnp.broadcast_to(b[None, :], (8, D)))


def _comb_body(norm, h_ref, s_ref, d_ref, ws_ref, wn_ref, b_ref, o_ref):
    deg = jnp.maximum(d_ref[...][:, :1], 1.0)
    hn = s_ref[...] / deg
    y = (jnp.dot(h_ref[...], ws_ref[...], preferred_element_type=jnp.float32,
                 precision=lax.Precision.HIGHEST)
         + jnp.dot(hn, wn_ref[...], preferred_element_type=jnp.float32,
                   precision=lax.Precision.HIGHEST)
         + b_ref[0:1, :])
    y = jnp.maximum(y, 0.0)
    if norm:
        nrm = jnp.sqrt(jnp.sum(y * y, axis=1, keepdims=True))
        y = y / jnp.maximum(nrm, 1e-12)
    o_ref[...] = y


def _comb(h, summ, deg, ws, wn, b, norm):
    return pl.pallas_call(
        functools.partial(_comb_body, norm),
        grid=(N // _BLK,),
        in_specs=[pl.BlockSpec((_BLK, D), lambda i: (i, 0)),
                  pl.BlockSpec((_BLK, D), lambda i: (i, 0)),
                  pl.BlockSpec((_BLK, DEGW), lambda i: (i, 0)),
                  pl.BlockSpec((D, D), lambda i: (0, 0)),
                  pl.BlockSpec((D, D), lambda i: (0, 0)),
                  pl.BlockSpec((8, D), lambda i: (0, 0))],
        out_specs=pl.BlockSpec((_BLK, D), lambda i: (i, 0)),
        out_shape=jax.ShapeDtypeStruct((N, D), jnp.float32),
    )(h, summ, deg, ws, wn, jnp.broadcast_to(b[None, :], (8, D)))


def kernel(x_user, x_item, ei_clicks, ei_clicked_by, Wp_user, bp_user,
           Wp_item, bp_item, Ws0_clicks, Wn0_clicks, b0_clicks, Ws0_cb,
           Wn0_cb, b0_cb, Ws1_clicks, Wn1_clicks, b1_clicks, Ws1_cb,
           Wn1_cb, b1_cb):
    ei_ck = ei_clicks.astype(jnp.int32)
    ei_cb = ei_clicked_by.astype(jnp.int32)
    src_ck, dst_ck = ei_ck[0], ei_ck[1]
    src_cb, dst_cb = ei_cb[0], ei_cb[1]
    zD = jnp.zeros((RCH, D), jnp.float32)
    zW = jnp.zeros((RCH, DEGW), jnp.float32)
    onesW = jnp.ones((CHUNK, DEGW), jnp.float32)

    h_u = _proj(x_user, Wp_user, bp_user)
    h_i = _proj(x_item, Wp_item, bp_item)

    deg_u, deg_i = _sc_deg(dst_ck, dst_cb, zW, onesW)
    summ_u, summ_i = _sc_agg(h_u, h_i, src_ck, dst_ck, src_cb, dst_cb, zD)
    h_u = _comb(h_u, summ_u, deg_u, Ws0_cb, Wn0_cb, b0_cb, norm=False)
    h_i = _comb(h_i, summ_i, deg_i, Ws0_clicks, Wn0_clicks, b0_clicks,
                norm=False)

    summ_u, summ_i = _sc_agg(h_u, h_i, src_ck, dst_ck, src_cb, dst_cb, zD)
    h_u = _comb(h_u, summ_u, deg_u, Ws1_cb, Wn1_cb, b1_cb, norm=True)
    h_i = _comb(h_i, summ_i, deg_i, Ws1_clicks, Wn1_clicks, b1_clicks,
                norm=True)
    return (h_u, h_i)


# speedup vs baseline: 5.7430x; 1.0197x over previous
import functools

import jax
import jax.numpy as jnp
from jax import lax
from jax.experimental import pallas as pl
from jax.experimental.pallas import tpu as pltpu
from jax.experimental.pallas import tpu_sc as plsc

N = 10000
E = 320000
D = 128
NS = 16
CHUNK = 96
EPP = E // NS
NCH = EPP // CHUNK
TCH = EPP - NCH * CHUNK
NPAIR = NCH // 2
RPT = 640
RCH = 160
DEGW = 128

_MESH = dict(core_axis_name="c", subcore_axis_name="s")


def _make_sc_agg():
    outs = (jax.ShapeDtypeStruct((N, D), jnp.float32),
            jax.ShapeDtypeStruct((N, D), jnp.float32))
    scratch = [
        pltpu.VMEM((CHUNK,), jnp.int32),
        pltpu.VMEM((CHUNK,), jnp.int32),
        pltpu.VMEM((CHUNK,), jnp.int32),
        pltpu.VMEM((CHUNK,), jnp.int32),
        pltpu.VMEM((CHUNK, D), jnp.float32),
        pltpu.VMEM((CHUNK, D), jnp.float32),
        pltpu.VMEM((RCH, D), jnp.float32),
        pltpu.VMEM((TCH,), jnp.int32),
        pltpu.VMEM((TCH,), jnp.int32),
        pltpu.VMEM_SHARED((N, D), jnp.float32),
        pltpu.SemaphoreType.DMA,
        pltpu.SemaphoreType.DMA,
        pltpu.SemaphoreType.DMA,
        pltpu.SemaphoreType.DMA,
    ]

    @functools.partial(pl.kernel, out_type=outs,
                       mesh=plsc.VectorSubcoreMesh(**_MESH),
                       scratch_types=scratch)
    def sc_agg(hu, hi, src_ck, dst_ck, src_cb, dst_cb, zD,
               summ_u, summ_i,
               is0, is1, id0, id1, rows0, rows1, stage_v, ist, idt, acc_sh,
               semi0, semi1, semg0, semg1):
        c = lax.axis_index("c")
        s = lax.axis_index("s")
        r0 = jnp.minimum(s * RPT, N - RPT)
        e0 = s * EPP

        def do_rel(feat_hbm, src_hbm, dst_hbm, summ_hbm):
            pltpu.sync_copy(zD, stage_v)

            def zero_chunk(k, carry):
                pltpu.sync_copy(stage_v, acc_sh.at[pl.ds(r0 + k * RCH, RCH), :])
                return carry

            lax.fori_loop(0, RPT // RCH, zero_chunk, 0)
            plsc.subcore_barrier()

            def idx_start(ci, bs, bd, sem):
                base = e0 + ci * CHUNK
                pltpu.async_copy(src_hbm.at[pl.ds(base, CHUNK)], bs, sem)
                pltpu.async_copy(dst_hbm.at[pl.ds(base, CHUNK)], bd, sem)

            def idx_wait(ci, bs, bd, sem):
                base = e0 + ci * CHUNK
                pltpu.make_async_copy(src_hbm.at[pl.ds(base, CHUNK)], bs,
                                      sem).wait()
                pltpu.make_async_copy(dst_hbm.at[pl.ds(base, CHUNK)], bd,
                                      sem).wait()

            def gather_start(bs, rows, sem):
                pltpu.async_copy(feat_hbm.at[bs], rows, sem)

            def gather_wait(bs, rows, sem):
                pltpu.make_async_copy(feat_hbm.at[bs], rows, sem).wait()

            def scatter(bd, rows):
                pltpu.sync_copy(rows, acc_sh.at[bd], add=True)

            idx_start(0, is0, id0, semi0)
            idx_start(1, is1, id1, semi1)
            idx_wait(0, is0, id0, semi0)
            gather_start(is0, rows0, semg0)
            idx_wait(1, is1, id1, semi1)

            def pair(j, carry):
                c0 = 2 * j
                gather_start(is1, rows1, semg1)
                gather_wait(is0, rows0, semg0)
                scatter(id0, rows0)
                idx_start(c0 + 2, is0, id0, semi0)
                gather_wait(is1, rows1, semg1)
                scatter(id1, rows1)
                idx_start(c0 + 3, is1, id1, semi1)
                idx_wait(c0 + 2, is0, id0, semi0)
                gather_start(is0, rows0, semg0)
                idx_wait(c0 + 3, is1, id1, semi1)
                return carry

            lax.fori_loop(0, NPAIR - 1, pair, 0)
            gather_start(is1, rows1, semg1)
            gather_wait(is0, rows0, semg0)
            scatter(id0, rows0)
            gather_wait(is1, rows1, semg1)
            scatter(id1, rows1)

            tbase = e0 + NCH * CHUNK
            pltpu.sync_copy(src_hbm.at[pl.ds(tbase, TCH)], ist)
            pltpu.sync_copy(dst_hbm.at[pl.ds(tbase, TCH)], idt)
            rows_t = rows0.at[pl.ds(0, TCH), :]
            pltpu.async_copy(feat_hbm.at[ist], rows_t, semg0).wait()
            pltpu.sync_copy(rows_t, acc_sh.at[idt], add=True)

            plsc.subcore_barrier()

            def out_chunk(k, carry):
                rb = r0 + k * RCH
                pltpu.sync_copy(acc_sh.at[pl.ds(rb, RCH), :], stage_v)
                pltpu.sync_copy(stage_v, summ_hbm.at[pl.ds(rb, RCH), :])
                return carry

            lax.fori_loop(0, RPT // RCH, out_chunk, 0)

        @pl.when(c == 0)
        def _():
            do_rel(hi, src_cb, dst_cb, summ_u)

        @pl.when(c == 1)
        def _():
            do_rel(hu, src_ck, dst_ck, summ_i)

    return sc_agg


_sc_agg = _make_sc_agg()


def _make_sc_deg():
    outs = (jax.ShapeDtypeStruct((N, DEGW), jnp.float32),
            jax.ShapeDtypeStruct((N, DEGW), jnp.float32))
    scratch = [
        pltpu.VMEM((CHUNK,), jnp.int32),
        pltpu.VMEM((CHUNK,), jnp.int32),
        pltpu.VMEM((CHUNK, DEGW), jnp.float32),
        pltpu.VMEM((RCH, DEGW), jnp.float32),
        pltpu.VMEM((TCH,), jnp.int32),
        pltpu.VMEM_SHARED((N, DEGW), jnp.float32),
        pltpu.SemaphoreType.DMA,
        pltpu.SemaphoreType.DMA,
    ]

    @functools.partial(pl.kernel, out_type=outs,
                       mesh=plsc.VectorSubcoreMesh(**_MESH),
                       scratch_types=scratch)
    def sc_deg(dst_ck, dst_cb, zW, onesW, deg_u, deg_i,
               id0, id1, ones_v, stageW_v, idt, degacc_sh, semi0, semi1):
        c = lax.axis_index("c")
        s = lax.axis_index("s")
        r0 = jnp.minimum(s * RPT, N - RPT)
        e0 = s * EPP

        def do_rel(dst_hbm, deg_hbm):
            pltpu.sync_copy(zW, stageW_v)
            pltpu.sync_copy(onesW, ones_v)

            def zero_chunk(k, carry):
                pltpu.sync_copy(stageW_v,
                                degacc_sh.at[pl.ds(r0 + k * RCH, RCH), :])
                return carry

            lax.fori_loop(0, RPT // RCH, zero_chunk, 0)
            plsc.subcore_barrier()

            def idx_start(ci, bd, sem):
                pltpu.async_copy(dst_hbm.at[pl.ds(e0 + ci * CHUNK, CHUNK)],
                                 bd, sem)

            def idx_wait(ci, bd, sem):
                pltpu.make_async_copy(
                    dst_hbm.at[pl.ds(e0 + ci * CHUNK, CHUNK)], bd, sem).wait()

            def scatter(bd):
                pltpu.sync_copy(ones_v, degacc_sh.at[bd], add=True)

            idx_start(0, id0, semi0)
            idx_start(1, id1, semi1)
            idx_wait(0, id0, semi0)
            idx_wait(1, id1, semi1)

            def pair(j, carry):
                c0 = 2 * j
                scatter(id0)
                idx_start(c0 + 2, id0, semi0)
                scatter(id1)
                idx_start(c0 + 3, id1, semi1)
                idx_wait(c0 + 2, id0, semi0)
                idx_wait(c0 + 3, id1, semi1)
                return carry

            lax.fori_loop(0, NPAIR - 1, pair, 0)
            scatter(id0)
            scatter(id1)
            tbase = e0 + NCH * CHUNK
            pltpu.sync_copy(dst_hbm.at[pl.ds(tbase, TCH)], idt)
            pltpu.sync_copy(ones_v.at[pl.ds(0, TCH), :],
                            degacc_sh.at[idt], add=True)
            plsc.subcore_barrier()

            def out_chunk(k, carry):
                rb = r0 + k * RCH
                pltpu.sync_copy(degacc_sh.at[pl.ds(rb, RCH), :], stageW_v)
                pltpu.sync_copy(stageW_v, deg_hbm.at[pl.ds(rb, RCH), :])
                return carry

            lax.fori_loop(0, RPT // RCH, out_chunk, 0)

        @pl.when(c == 0)
        def _():
            do_rel(dst_cb, deg_u)

        @pl.when(c == 1)
        def _():
            do_rel(dst_ck, deg_i)

    return sc_deg


_sc_deg = _make_sc_deg()


_BLK = 1000

_ROWSPEC = pl.BlockSpec((_BLK, D), lambda i: (i, 0))
_WSPEC = pl.BlockSpec((D, D), lambda i: (0, 0))
_BSPEC = pl.BlockSpec((8, D), lambda i: (0, 0))


def _proj_body(xu_ref, xi_ref, wu_ref, bu_ref, wi_ref, bi_ref,
               ou_ref, oi_ref):
    yu = jnp.dot(xu_ref[...], wu_ref[...], preferred_element_type=jnp.float32,
                 precision=lax.Precision.HIGHEST)
    ou_ref[...] = jnp.maximum(yu + bu_ref[0:1, :], 0.0)
    yi = jnp.dot(xi_ref[...], wi_ref[...], preferred_element_type=jnp.float32,
                 precision=lax.Precision.HIGHEST)
    oi_ref[...] = jnp.maximum(yi + bi_ref[0:1, :], 0.0)


def _proj2(xu, xi, wu, bu, wi, bi):
    return pl.pallas_call(
        _proj_body,
        grid=(N // _BLK,),
        in_specs=[_ROWSPEC, _ROWSPEC, _WSPEC, _BSPEC, _WSPEC, _BSPEC],
        out_specs=(_ROWSPEC, _ROWSPEC),
        out_shape=(jax.ShapeDtypeStruct((N, D), jnp.float32),
                   jax.ShapeDtypeStruct((N, D), jnp.float32)),
    )(xu, xi, wu, jnp.broadcast_to(bu[None, :], (8, D)),
      wi, jnp.broadcast_to(bi[None, :], (8, D)))


def _comb_half(h, summ, deg, ws, wn, b):
    dg = jnp.maximum(deg[:, :1], 1.0)
    hn = summ / dg
    return (jnp.dot(h, ws, preferred_element_type=jnp.float32,
                    precision=lax.Precision.HIGHEST)
            + jnp.dot(hn, wn, preferred_element_type=jnp.float32,
                      precision=lax.Precision.HIGHEST)
            + b[0:1, :])


def _comb_body(norm, hu_ref, su_ref, du_ref, wsu_ref, wnu_ref, bu_ref,
               hi_ref, si_ref, di_ref, wsi_ref, wni_ref, bi_ref,
               ou_ref, oi_ref):
    for h_ref, s_ref, d_ref, ws_ref, wn_ref, b_ref, o_ref in (
            (hu_ref, su_ref, du_ref, wsu_ref, wnu_ref, bu_ref, ou_ref),
            (hi_ref, si_ref, di_ref, wsi_ref, wni_ref, bi_ref, oi_ref)):
        y = _comb_half(h_ref[...], s_ref[...], d_ref[...], ws_ref[...],
                       wn_ref[...], b_ref[...])
        y = jnp.maximum(y, 0.0)
        if norm:
            nrm = jnp.sqrt(jnp.sum(y * y, axis=1, keepdims=True))
            y = y / jnp.maximum(nrm, 1e-12)
        o_ref[...] = y


def _comb2(hu, su, du, wsu, wnu, bu, hi, si, di, wsi, wni, bi, norm):
    half = [_ROWSPEC, _ROWSPEC, pl.BlockSpec((_BLK, DEGW), lambda i: (i, 0)),
            _WSPEC, _WSPEC, _BSPEC]
    return pl.pallas_call(
        functools.partial(_comb_body, norm),
        grid=(N // _BLK,),
        in_specs=half + half,
        out_specs=(_ROWSPEC, _ROWSPEC),
        out_shape=(jax.ShapeDtypeStruct((N, D), jnp.float32),
                   jax.ShapeDtypeStruct((N, D), jnp.float32)),
    )(hu, su, du, wsu, wnu, jnp.broadcast_to(bu[None, :], (8, D)),
      hi, si, di, wsi, wni, jnp.broadcast_to(bi[None, :], (8, D)))


def kernel(x_user, x_item, ei_clicks, ei_clicked_by, Wp_user, bp_user,
           Wp_item, bp_item, Ws0_clicks, Wn0_clicks, b0_clicks, Ws0_cb,
           Wn0_cb, b0_cb, Ws1_clicks, Wn1_clicks, b1_clicks, Ws1_cb,
           Wn1_cb, b1_cb):
    ei_ck = ei_clicks.astype(jnp.int32)
    ei_cb = ei_clicked_by.astype(jnp.int32)
    src_ck, dst_ck = ei_ck[0], ei_ck[1]
    src_cb, dst_cb = ei_cb[0], ei_cb[1]
    zD = jnp.zeros((RCH, D), jnp.float32)
    zW = jnp.zeros((RCH, DEGW), jnp.float32)
    onesW = jnp.ones((CHUNK, DEGW), jnp.float32)

    h_u, h_i = _proj2(x_user, x_item, Wp_user, bp_user, Wp_item, bp_item)

    deg_u, deg_i = _sc_deg(dst_ck, dst_cb, zW, onesW)
    summ_u, summ_i = _sc_agg(h_u, h_i, src_ck, dst_ck, src_cb, dst_cb, zD)
    h_u, h_i = _comb2(h_u, summ_u, deg_u, Ws0_cb, Wn0_cb, b0_cb,
                      h_i, summ_i, deg_i, Ws0_clicks, Wn0_clicks, b0_clicks,
                      norm=False)

    summ_u, summ_i = _sc_agg(h_u, h_i, src_ck, dst_ck, src_cb, dst_cb, zD)
    h_u, h_i = _comb2(h_u, summ_u, deg_u, Ws1_cb, Wn1_cb, b1_cb,
                      h_i, summ_i, deg_i, Ws1_clicks, Wn1_clicks, b1_clicks,
                      norm=True)
    return (h_u, h_i)


# speedup vs baseline: 5.7666x; 1.0041x over previous
import functools

import jax
import jax.numpy as jnp
from jax import lax
from jax.experimental import pallas as pl
from jax.experimental.pallas import tpu as pltpu
from jax.experimental.pallas import tpu_sc as plsc

N = 10000
E = 320000
D = 128
NS = 16
CHUNK = 96
EPP = E // NS
NCH = EPP // CHUNK
TCH = EPP - NCH * CHUNK
NPAIR = NCH // 2
RPT = 640
RCH = 160
DEGW = 128

_MESH = dict(core_axis_name="c", subcore_axis_name="s")


def _make_sc_agg(with_deg):
    outs = (jax.ShapeDtypeStruct((N, D), jnp.float32),
            jax.ShapeDtypeStruct((N, D), jnp.float32))
    if with_deg:
        outs += (jax.ShapeDtypeStruct((N, D), jnp.float32),
                 jax.ShapeDtypeStruct((N, D), jnp.float32))
    scratch = [
        pltpu.VMEM((CHUNK,), jnp.int32),
        pltpu.VMEM((CHUNK,), jnp.int32),
        pltpu.VMEM((CHUNK,), jnp.int32),
        pltpu.VMEM((CHUNK,), jnp.int32),
        pltpu.VMEM((CHUNK, D), jnp.float32),
        pltpu.VMEM((CHUNK, D), jnp.float32),
        pltpu.VMEM((RCH, D), jnp.float32),
        pltpu.VMEM((TCH,), jnp.int32),
        pltpu.VMEM((TCH,), jnp.int32),
        pltpu.VMEM_SHARED((N, D), jnp.float32),
        pltpu.SemaphoreType.DMA,
        pltpu.SemaphoreType.DMA,
        pltpu.SemaphoreType.DMA,
        pltpu.SemaphoreType.DMA,
    ]

    @functools.partial(pl.kernel, out_type=outs,
                       mesh=plsc.VectorSubcoreMesh(**_MESH),
                       scratch_types=scratch)
    def sc_agg(*refs):
        if with_deg:
            (hu, hi, src_ck, dst_ck, src_cb, dst_cb, zD, onesW,
             summ_u, summ_i, deg_u, deg_i,
             is0, is1, id0, id1, rows0, rows1, stage_v, ist, idt, acc_sh,
             semi0, semi1, semg0, semg1) = refs
        else:
            (hu, hi, src_ck, dst_ck, src_cb, dst_cb, zD,
             summ_u, summ_i,
             is0, is1, id0, id1, rows0, rows1, stage_v, ist, idt, acc_sh,
             semi0, semi1, semg0, semg1) = refs
            deg_u = deg_i = None
        c = lax.axis_index("c")
        s = lax.axis_index("s")
        r0 = jnp.minimum(s * RPT, N - RPT)
        e0 = s * EPP

        def do_rel(feat_hbm, src_hbm, dst_hbm, summ_hbm, deg_hbm):
            pltpu.sync_copy(zD, stage_v)

            def zero_chunk(k, carry):
                pltpu.sync_copy(stage_v, acc_sh.at[pl.ds(r0 + k * RCH, RCH), :])
                return carry

            lax.fori_loop(0, RPT // RCH, zero_chunk, 0)
            plsc.subcore_barrier()

            def idx_start(ci, bs, bd, sem):
                base = e0 + ci * CHUNK
                pltpu.async_copy(src_hbm.at[pl.ds(base, CHUNK)], bs, sem)
                pltpu.async_copy(dst_hbm.at[pl.ds(base, CHUNK)], bd, sem)

            def idx_wait(ci, bs, bd, sem):
                base = e0 + ci * CHUNK
                pltpu.make_async_copy(src_hbm.at[pl.ds(base, CHUNK)], bs,
                                      sem).wait()
                pltpu.make_async_copy(dst_hbm.at[pl.ds(base, CHUNK)], bd,
                                      sem).wait()

            def gather_start(bs, rows, sem):
                pltpu.async_copy(feat_hbm.at[bs], rows, sem)

            def gather_wait(bs, rows, sem):
                pltpu.make_async_copy(feat_hbm.at[bs], rows, sem).wait()

            def scatter(bd, rows):
                pltpu.sync_copy(rows, acc_sh.at[bd], add=True)

            idx_start(0, is0, id0, semi0)
            idx_start(1, is1, id1, semi1)
            idx_wait(0, is0, id0, semi0)
            gather_start(is0, rows0, semg0)
            idx_wait(1, is1, id1, semi1)

            def pair(j, carry):
                c0 = 2 * j
                gather_start(is1, rows1, semg1)
                gather_wait(is0, rows0, semg0)
                scatter(id0, rows0)
                idx_start(c0 + 2, is0, id0, semi0)
                gather_wait(is1, rows1, semg1)
                scatter(id1, rows1)
                idx_start(c0 + 3, is1, id1, semi1)
                idx_wait(c0 + 2, is0, id0, semi0)
                gather_start(is0, rows0, semg0)
                idx_wait(c0 + 3, is1, id1, semi1)
                return carry

            lax.fori_loop(0, NPAIR - 1, pair, 0)
            gather_start(is1, rows1, semg1)
            gather_wait(is0, rows0, semg0)
            scatter(id0, rows0)
            gather_wait(is1, rows1, semg1)
            scatter(id1, rows1)

            tbase = e0 + NCH * CHUNK
            pltpu.sync_copy(src_hbm.at[pl.ds(tbase, TCH)], ist)
            pltpu.sync_copy(dst_hbm.at[pl.ds(tbase, TCH)], idt)
            rows_t = rows0.at[pl.ds(0, TCH), :]
            pltpu.async_copy(feat_hbm.at[ist], rows_t, semg0).wait()
            pltpu.sync_copy(rows_t, acc_sh.at[idt], add=True)

            plsc.subcore_barrier()

            def out_chunk(k, carry):
                rb = r0 + k * RCH
                pltpu.sync_copy(acc_sh.at[pl.ds(rb, RCH), :], stage_v)
                pltpu.sync_copy(stage_v, summ_hbm.at[pl.ds(rb, RCH), :])
                return carry

            lax.fori_loop(0, RPT // RCH, out_chunk, 0)

            if with_deg:
                pltpu.sync_copy(zD, stage_v)
                lax.fori_loop(0, RPT // RCH, zero_chunk, 0)
                pltpu.sync_copy(onesW, rows0)
                plsc.subcore_barrier()

                def dscatter(bd):
                    pltpu.sync_copy(rows0, acc_sh.at[bd], add=True)

                def didx_start(ci, bd, sem):
                    pltpu.async_copy(
                        dst_hbm.at[pl.ds(e0 + ci * CHUNK, CHUNK)], bd, sem)

                def didx_wait(ci, bd, sem):
                    pltpu.make_async_copy(
                        dst_hbm.at[pl.ds(e0 + ci * CHUNK, CHUNK)], bd,
                        sem).wait()

                didx_start(0, id0, semi0)
                didx_start(1, id1, semi1)
                didx_wait(0, id0, semi0)
                didx_wait(1, id1, semi1)

                def dpair(j, carry):
                    c0 = 2 * j
                    dscatter(id0)
                    didx_start(c0 + 2, id0, semi0)
                    dscatter(id1)
                    didx_start(c0 + 3, id1, semi1)
                    didx_wait(c0 + 2, id0, semi0)
                    didx_wait(c0 + 3, id1, semi1)
                    return carry

                lax.fori_loop(0, NPAIR - 1, dpair, 0)
                dscatter(id0)
                dscatter(id1)
                tb = e0 + NCH * CHUNK
                pltpu.sync_copy(dst_hbm.at[pl.ds(tb, TCH)], idt)
                pltpu.sync_copy(rows0.at[pl.ds(0, TCH), :],
                                acc_sh.at[idt], add=True)
                plsc.subcore_barrier()

                def dout_chunk(k, carry):
                    rb = r0 + k * RCH
                    pltpu.sync_copy(acc_sh.at[pl.ds(rb, RCH), :], stage_v)
                    pltpu.sync_copy(stage_v, deg_hbm.at[pl.ds(rb, RCH), :])
                    return carry

                lax.fori_loop(0, RPT // RCH, dout_chunk, 0)

        @pl.when(c == 0)
        def _():
            do_rel(hi, src_cb, dst_cb, summ_u, deg_u)

        @pl.when(c == 1)
        def _():
            do_rel(hu, src_ck, dst_ck, summ_i, deg_i)

    return sc_agg


_sc_agg0 = _make_sc_agg(True)
_sc_agg1 = _make_sc_agg(False)


_BLK = 1000

_ROWSPEC = pl.BlockSpec((_BLK, D), lambda i: (i, 0))
_WSPEC = pl.BlockSpec((D, D), lambda i: (0, 0))
_BSPEC = pl.BlockSpec((8, D), lambda i: (0, 0))


def _proj_body(xu_ref, xi_ref, wu_ref, bu_ref, wi_ref, bi_ref,
               ou_ref, oi_ref):
    yu = jnp.dot(xu_ref[...], wu_ref[...], preferred_element_type=jnp.float32,
                 precision=lax.Precision.HIGHEST)
    ou_ref[...] = jnp.maximum(yu + bu_ref[0:1, :], 0.0)
    yi = jnp.dot(xi_ref[...], wi_ref[...], preferred_element_type=jnp.float32,
                 precision=lax.Precision.HIGHEST)
    oi_ref[...] = jnp.maximum(yi + bi_ref[0:1, :], 0.0)


def _proj2(xu, xi, wu, bu, wi, bi):
    return pl.pallas_call(
        _proj_body,
        grid=(N // _BLK,),
        in_specs=[_ROWSPEC, _ROWSPEC, _WSPEC, _BSPEC, _WSPEC, _BSPEC],
        out_specs=(_ROWSPEC, _ROWSPEC),
        out_shape=(jax.ShapeDtypeStruct((N, D), jnp.float32),
                   jax.ShapeDtypeStruct((N, D), jnp.float32)),
    )(xu, xi, wu, jnp.broadcast_to(bu[None, :], (8, D)),
      wi, jnp.broadcast_to(bi[None, :], (8, D)))


def _comb_half(h, summ, deg, ws, wn, b):
    dg = jnp.maximum(deg[:, :1], 1.0)
    hn = summ / dg
    return (jnp.dot(h, ws, preferred_element_type=jnp.float32,
                    precision=lax.Precision.HIGHEST)
            + jnp.dot(hn, wn, preferred_element_type=jnp.float32,
                      precision=lax.Precision.HIGHEST)
            + b[0:1, :])


def _comb_body(norm, hu_ref, su_ref, du_ref, wsu_ref, wnu_ref, bu_ref,
               hi_ref, si_ref, di_ref, wsi_ref, wni_ref, bi_ref,
               ou_ref, oi_ref):
    for h_ref, s_ref, d_ref, ws_ref, wn_ref, b_ref, o_ref in (
            (hu_ref, su_ref, du_ref, wsu_ref, wnu_ref, bu_ref, ou_ref),
            (hi_ref, si_ref, di_ref, wsi_ref, wni_ref, bi_ref, oi_ref)):
        y = _comb_half(h_ref[...], s_ref[...], d_ref[...], ws_ref[...],
                       wn_ref[...], b_ref[...])
        y = jnp.maximum(y, 0.0)
        if norm:
            nrm = jnp.sqrt(jnp.sum(y * y, axis=1, keepdims=True))
            y = y / jnp.maximum(nrm, 1e-12)
        o_ref[...] = y


def _comb2(hu, su, du, wsu, wnu, bu, hi, si, di, wsi, wni, bi, norm):
    half = [_ROWSPEC, _ROWSPEC, pl.BlockSpec((_BLK, DEGW), lambda i: (i, 0)),
            _WSPEC, _WSPEC, _BSPEC]
    return pl.pallas_call(
        functools.partial(_comb_body, norm),
        grid=(N // _BLK,),
        in_specs=half + half,
        out_specs=(_ROWSPEC, _ROWSPEC),
        out_shape=(jax.ShapeDtypeStruct((N, D), jnp.float32),
                   jax.ShapeDtypeStruct((N, D), jnp.float32)),
    )(hu, su, du, wsu, wnu, jnp.broadcast_to(bu[None, :], (8, D)),
      hi, si, di, wsi, wni, jnp.broadcast_to(bi[None, :], (8, D)))


def kernel(x_user, x_item, ei_clicks, ei_clicked_by, Wp_user, bp_user,
           Wp_item, bp_item, Ws0_clicks, Wn0_clicks, b0_clicks, Ws0_cb,
           Wn0_cb, b0_cb, Ws1_clicks, Wn1_clicks, b1_clicks, Ws1_cb,
           Wn1_cb, b1_cb):
    ei_ck = ei_clicks.astype(jnp.int32)
    ei_cb = ei_clicked_by.astype(jnp.int32)
    src_ck, dst_ck = ei_ck[0], ei_ck[1]
    src_cb, dst_cb = ei_cb[0], ei_cb[1]
    zD = jnp.zeros((RCH, D), jnp.float32)
    onesW = jnp.ones((CHUNK, D), jnp.float32)

    h_u, h_i = _proj2(x_user, x_item, Wp_user, bp_user, Wp_item, bp_item)

    summ_u, summ_i, deg_u, deg_i = _sc_agg0(h_u, h_i, src_ck, dst_ck,
                                            src_cb, dst_cb, zD, onesW)
    h_u, h_i = _comb2(h_u, summ_u, deg_u, Ws0_cb, Wn0_cb, b0_cb,
                      h_i, summ_i, deg_i, Ws0_clicks, Wn0_clicks, b0_clicks,
                      norm=False)

    summ_u, summ_i = _sc_agg1(h_u, h_i, src_ck, dst_ck, src_cb, dst_cb, zD)
    h_u, h_i = _comb2(h_u, summ_u, deg_u, Ws1_cb, Wn1_cb, b1_cb,
                      h_i, summ_i, deg_i, Ws1_clicks, Wn1_clicks, b1_clicks,
                      norm=True)
    return (h_u, h_i)


# speedup vs baseline: 5.8935x; 1.0220x over previous
import functools

import jax
import jax.numpy as jnp
from jax import lax
from jax.experimental import pallas as pl
from jax.experimental.pallas import tpu as pltpu
from jax.experimental.pallas import tpu_sc as plsc

N = 10000
E = 320000
D = 128
NS = 16
CHUNK = 112
EPP = E // NS
NCH = EPP // CHUNK
TCH = EPP - NCH * CHUNK
NPAIR = NCH // 2
RPT = 640
RCH = 160
DEGW = 128

_MESH = dict(core_axis_name="c", subcore_axis_name="s")


def _make_sc_agg(with_deg):
    outs = (jax.ShapeDtypeStruct((N, D), jnp.float32),
            jax.ShapeDtypeStruct((N, D), jnp.float32))
    if with_deg:
        outs += (jax.ShapeDtypeStruct((N, D), jnp.float32),
                 jax.ShapeDtypeStruct((N, D), jnp.float32))
    scratch = [
        pltpu.VMEM((CHUNK,), jnp.int32),
        pltpu.VMEM((CHUNK,), jnp.int32),
        pltpu.VMEM((CHUNK,), jnp.int32),
        pltpu.VMEM((CHUNK,), jnp.int32),
        pltpu.VMEM((CHUNK, D), jnp.float32),
        pltpu.VMEM((CHUNK, D), jnp.float32),
        pltpu.VMEM((RCH, D), jnp.float32),
        pltpu.VMEM((TCH,), jnp.int32),
        pltpu.VMEM((TCH,), jnp.int32),
        pltpu.VMEM_SHARED((N, D), jnp.float32),
        pltpu.SemaphoreType.DMA,
        pltpu.SemaphoreType.DMA,
        pltpu.SemaphoreType.DMA,
        pltpu.SemaphoreType.DMA,
    ]

    @functools.partial(pl.kernel, out_type=outs,
                       mesh=plsc.VectorSubcoreMesh(**_MESH),
                       scratch_types=scratch)
    def sc_agg(*refs):
        if with_deg:
            (hu, hi, src_ck, dst_ck, src_cb, dst_cb, zD, onesW,
             summ_u, summ_i, deg_u, deg_i,
             is0, is1, id0, id1, rows0, rows1, stage_v, ist, idt, acc_sh,
             semi0, semi1, semg0, semg1) = refs
        else:
            (hu, hi, src_ck, dst_ck, src_cb, dst_cb, zD,
             summ_u, summ_i,
             is0, is1, id0, id1, rows0, rows1, stage_v, ist, idt, acc_sh,
             semi0, semi1, semg0, semg1) = refs
            deg_u = deg_i = None
        c = lax.axis_index("c")
        s = lax.axis_index("s")
        r0 = jnp.minimum(s * RPT, N - RPT)
        e0 = s * EPP

        def do_rel(feat_hbm, src_hbm, dst_hbm, summ_hbm, deg_hbm):
            pltpu.sync_copy(zD, stage_v)

            def zero_chunk(k, carry):
                pltpu.sync_copy(stage_v, acc_sh.at[pl.ds(r0 + k * RCH, RCH), :])
                return carry

            lax.fori_loop(0, RPT // RCH, zero_chunk, 0)
            plsc.subcore_barrier()

            def idx_start(ci, bs, bd, sem):
                base = e0 + ci * CHUNK
                pltpu.async_copy(src_hbm.at[pl.ds(base, CHUNK)], bs, sem)
                pltpu.async_copy(dst_hbm.at[pl.ds(base, CHUNK)], bd, sem)

            def idx_wait(ci, bs, bd, sem):
                base = e0 + ci * CHUNK
                pltpu.make_async_copy(src_hbm.at[pl.ds(base, CHUNK)], bs,
                                      sem).wait()
                pltpu.make_async_copy(dst_hbm.at[pl.ds(base, CHUNK)], bd,
                                      sem).wait()

            def gather_start(bs, rows, sem):
                pltpu.async_copy(feat_hbm.at[bs], rows, sem)

            def gather_wait(bs, rows, sem):
                pltpu.make_async_copy(feat_hbm.at[bs], rows, sem).wait()

            def scatter(bd, rows):
                pltpu.sync_copy(rows, acc_sh.at[bd], add=True)

            idx_start(0, is0, id0, semi0)
            idx_start(1, is1, id1, semi1)
            idx_wait(0, is0, id0, semi0)
            gather_start(is0, rows0, semg0)
            idx_wait(1, is1, id1, semi1)

            def pair(j, carry):
                c0 = 2 * j
                gather_start(is1, rows1, semg1)
                gather_wait(is0, rows0, semg0)
                scatter(id0, rows0)
                idx_start(c0 + 2, is0, id0, semi0)
                gather_wait(is1, rows1, semg1)
                scatter(id1, rows1)
                idx_start(c0 + 3, is1, id1, semi1)
                idx_wait(c0 + 2, is0, id0, semi0)
                gather_start(is0, rows0, semg0)
                idx_wait(c0 + 3, is1, id1, semi1)
                return carry

            lax.fori_loop(0, NPAIR - 1, pair, 0)
            gather_start(is1, rows1, semg1)
            gather_wait(is0, rows0, semg0)
            scatter(id0, rows0)
            gather_wait(is1, rows1, semg1)
            scatter(id1, rows1)

            tbase = e0 + NCH * CHUNK
            pltpu.sync_copy(src_hbm.at[pl.ds(tbase, TCH)], ist)
            pltpu.sync_copy(dst_hbm.at[pl.ds(tbase, TCH)], idt)
            rows_t = rows0.at[pl.ds(0, TCH), :]
            pltpu.async_copy(feat_hbm.at[ist], rows_t, semg0).wait()
            pltpu.sync_copy(rows_t, acc_sh.at[idt], add=True)

            plsc.subcore_barrier()

            def out_chunk(k, carry):
                rb = r0 + k * RCH
                pltpu.sync_copy(acc_sh.at[pl.ds(rb, RCH), :], stage_v)
                pltpu.sync_copy(stage_v, summ_hbm.at[pl.ds(rb, RCH), :])
                return carry

            lax.fori_loop(0, RPT // RCH, out_chunk, 0)

            if with_deg:
                pltpu.sync_copy(zD, stage_v)
                lax.fori_loop(0, RPT // RCH, zero_chunk, 0)
                pltpu.sync_copy(onesW, rows0)
                plsc.subcore_barrier()

                def dscatter(bd):
                    pltpu.sync_copy(rows0, acc_sh.at[bd], add=True)

                def didx_start(ci, bd, sem):
                    pltpu.async_copy(
                        dst_hbm.at[pl.ds(e0 + ci * CHUNK, CHUNK)], bd, sem)

                def didx_wait(ci, bd, sem):
                    pltpu.make_async_copy(
                        dst_hbm.at[pl.ds(e0 + ci * CHUNK, CHUNK)], bd,
                        sem).wait()

                didx_start(0, id0, semi0)
                didx_start(1, id1, semi1)
                didx_wait(0, id0, semi0)
                didx_wait(1, id1, semi1)

                def dpair(j, carry):
                    c0 = 2 * j
                    dscatter(id0)
                    didx_start(c0 + 2, id0, semi0)
                    dscatter(id1)
                    didx_start(c0 + 3, id1, semi1)
                    didx_wait(c0 + 2, id0, semi0)
                    didx_wait(c0 + 3, id1, semi1)
                    return carry

                lax.fori_loop(0, NPAIR - 1, dpair, 0)
                dscatter(id0)
                dscatter(id1)
                tb = e0 + NCH * CHUNK
                pltpu.sync_copy(dst_hbm.at[pl.ds(tb, TCH)], idt)
                pltpu.sync_copy(rows0.at[pl.ds(0, TCH), :],
                                acc_sh.at[idt], add=True)
                plsc.subcore_barrier()

                def dout_chunk(k, carry):
                    rb = r0 + k * RCH
                    pltpu.sync_copy(acc_sh.at[pl.ds(rb, RCH), :], stage_v)
                    pltpu.sync_copy(stage_v, deg_hbm.at[pl.ds(rb, RCH), :])
                    return carry

                lax.fori_loop(0, RPT // RCH, dout_chunk, 0)

        @pl.when(c == 0)
        def _():
            do_rel(hi, src_cb, dst_cb, summ_u, deg_u)

        @pl.when(c == 1)
        def _():
            do_rel(hu, src_ck, dst_ck, summ_i, deg_i)

    return sc_agg


_sc_agg0 = _make_sc_agg(True)
_sc_agg1 = _make_sc_agg(False)


_BLK = 1000

_ROWSPEC = pl.BlockSpec((_BLK, D), lambda i: (i, 0))
_WSPEC = pl.BlockSpec((D, D), lambda i: (0, 0))
_BSPEC = pl.BlockSpec((8, D), lambda i: (0, 0))


def _proj_body(xu_ref, xi_ref, wu_ref, bu_ref, wi_ref, bi_ref,
               ou_ref, oi_ref):
    yu = jnp.dot(xu_ref[...], wu_ref[...], preferred_element_type=jnp.float32,
                 precision=lax.Precision.HIGHEST)
    ou_ref[...] = jnp.maximum(yu + bu_ref[0:1, :], 0.0)
    yi = jnp.dot(xi_ref[...], wi_ref[...], preferred_element_type=jnp.float32,
                 precision=lax.Precision.HIGHEST)
    oi_ref[...] = jnp.maximum(yi + bi_ref[0:1, :], 0.0)


def _proj2(xu, xi, wu, bu, wi, bi):
    return pl.pallas_call(
        _proj_body,
        grid=(N // _BLK,),
        in_specs=[_ROWSPEC, _ROWSPEC, _WSPEC, _BSPEC, _WSPEC, _BSPEC],
        out_specs=(_ROWSPEC, _ROWSPEC),
        out_shape=(jax.ShapeDtypeStruct((N, D), jnp.float32),
                   jax.ShapeDtypeStruct((N, D), jnp.float32)),
    )(xu, xi, wu, jnp.broadcast_to(bu[None, :], (8, D)),
      wi, jnp.broadcast_to(bi[None, :], (8, D)))


def _comb_half(h, summ, deg, ws, wn, b):
    dg = jnp.maximum(deg[:, :1], 1.0)
    hn = summ / dg
    return (jnp.dot(h, ws, preferred_element_type=jnp.float32,
                    precision=lax.Precision.HIGHEST)
            + jnp.dot(hn, wn, preferred_element_type=jnp.float32,
                      precision=lax.Precision.HIGHEST)
            + b[0:1, :])


def _comb_body(norm, hu_ref, su_ref, du_ref, wsu_ref, wnu_ref, bu_ref,
               hi_ref, si_ref, di_ref, wsi_ref, wni_ref, bi_ref,
               ou_ref, oi_ref):
    for h_ref, s_ref, d_ref, ws_ref, wn_ref, b_ref, o_ref in (
            (hu_ref, su_ref, du_ref, wsu_ref, wnu_ref, bu_ref, ou_ref),
            (hi_ref, si_ref, di_ref, wsi_ref, wni_ref, bi_ref, oi_ref)):
        y = _comb_half(h_ref[...], s_ref[...], d_ref[...], ws_ref[...],
                       wn_ref[...], b_ref[...])
        y = jnp.maximum(y, 0.0)
        if norm:
            nrm = jnp.sqrt(jnp.sum(y * y, axis=1, keepdims=True))
            y = y / jnp.maximum(nrm, 1e-12)
        o_ref[...] = y


def _comb2(hu, su, du, wsu, wnu, bu, hi, si, di, wsi, wni, bi, norm):
    half = [_ROWSPEC, _ROWSPEC, pl.BlockSpec((_BLK, DEGW), lambda i: (i, 0)),
            _WSPEC, _WSPEC, _BSPEC]
    return pl.pallas_call(
        functools.partial(_comb_body, norm),
        grid=(N // _BLK,),
        in_specs=half + half,
        out_specs=(_ROWSPEC, _ROWSPEC),
        out_shape=(jax.ShapeDtypeStruct((N, D), jnp.float32),
                   jax.ShapeDtypeStruct((N, D), jnp.float32)),
    )(hu, su, du, wsu, wnu, jnp.broadcast_to(bu[None, :], (8, D)),
      hi, si, di, wsi, wni, jnp.broadcast_to(bi[None, :], (8, D)))


def kernel(x_user, x_item, ei_clicks, ei_clicked_by, Wp_user, bp_user,
           Wp_item, bp_item, Ws0_clicks, Wn0_clicks, b0_clicks, Ws0_cb,
           Wn0_cb, b0_cb, Ws1_clicks, Wn1_clicks, b1_clicks, Ws1_cb,
           Wn1_cb, b1_cb):
    ei_ck = ei_clicks.astype(jnp.int32)
    ei_cb = ei_clicked_by.astype(jnp.int32)
    src_ck, dst_ck = ei_ck[0], ei_ck[1]
    src_cb, dst_cb = ei_cb[0], ei_cb[1]
    zD = jnp.zeros((RCH, D), jnp.float32)
    onesW = jnp.ones((CHUNK, D), jnp.float32)

    h_u, h_i = _proj2(x_user, x_item, Wp_user, bp_user, Wp_item, bp_item)

    summ_u, summ_i, deg_u, deg_i = _sc_agg0(h_u, h_i, src_ck, dst_ck,
                                            src_cb, dst_cb, zD, onesW)
    h_u, h_i = _comb2(h_u, summ_u, deg_u, Ws0_cb, Wn0_cb, b0_cb,
                      h_i, summ_i, deg_i, Ws0_clicks, Wn0_clicks, b0_clicks,
                      norm=False)

    summ_u, summ_i = _sc_agg1(h_u, h_i, src_ck, dst_ck, src_cb, dst_cb, zD)
    h_u, h_i = _comb2(h_u, summ_u, deg_u, Ws1_cb, Wn1_cb, b1_cb,
                      h_i, summ_i, deg_i, Ws1_clicks, Wn1_clicks, b1_clicks,
                      norm=True)
    return (h_u, h_i)
